# Initial kernel scaffold; baseline (speedup 1.0000x reference)
import jax
import jax.numpy as jnp
from jax.experimental import pallas as pl


def kernel(x, pos_table):
    raise NotImplementedError("write your pallas kernel here")


# speedup vs baseline: 1.6710x; 1.6710x over previous
import jax
import jax.numpy as jnp
from jax.experimental import pallas as pl

_TB = 1024


def _body(x_ref, t_ref, o_ref):
    o_ref[...] = x_ref[...] + t_ref[...][None]


def kernel(x, pos_table):
    B, T, D = x.shape
    grid = (T // _TB, B)
    return pl.pallas_call(
        _body,
        grid=grid,
        in_specs=[
            pl.BlockSpec((1, _TB, D), lambda i, j: (j, i, 0)),
            pl.BlockSpec((_TB, D), lambda i, j: (i, 0)),
        ],
        out_specs=pl.BlockSpec((1, _TB, D), lambda i, j: (j, i, 0)),
        out_shape=jax.ShapeDtypeStruct((B, T, D), x.dtype),
    )(x, pos_table)
